# baseline (speedup 1.0000x reference)
import functools

import jax
import jax.numpy as jnp
from jax.experimental import pallas as pl
from jax.experimental.pallas import tpu as pltpu


def _fused_body(x_ref, w_ref, o_ref, *, c_in, c_out, h, w, inv_hw):
    xb = x_ref[0].reshape(c_in, h, w)
    s = jnp.sum(xb, axis=1)
    m = jnp.sum(s, axis=1, keepdims=True) * inv_hw
    y = jax.lax.dot_general(
        w_ref[...], m,
        dimension_numbers=(((1,), (0,)), ((), ())),
        preferred_element_type=jnp.float32,
    )
    y = jnp.maximum(y, 0.0)
    o_ref[0] = jnp.broadcast_to(y[:, :, None], (c_out, h, w)).reshape(c_out * h, w)


def kernel(x, weight):
    n, c_in, h, w = x.shape
    c_out = weight.shape[0]
    x_flat = x.reshape(n, c_in * h, w)
    w2d = weight.reshape(c_out, c_in)

    body = functools.partial(
        _fused_body, c_in=c_in, c_out=c_out, h=h, w=w,
        inv_hw=float(1.0 / (h * w)))

    out_flat = pl.pallas_call(
        body,
        out_shape=jax.ShapeDtypeStruct((n, c_out * h, w), x.dtype),
        grid=(n,),
        in_specs=[
            pl.BlockSpec((1, c_in * h, w), lambda i: (i, 0, 0)),
            pl.BlockSpec((c_out, c_in), lambda i: (0, 0)),
        ],
        out_specs=pl.BlockSpec((1, c_out * h, w), lambda i: (i, 0, 0)),
        compiler_params=pltpu.CompilerParams(
            dimension_semantics=("parallel",),
            vmem_limit_bytes=60 * 1024 * 1024,
        ),
    )(x_flat, w2d)
    return out_flat.reshape(n, c_out, h, w)
